# baseline (speedup 1.0000x reference)
import functools

import jax
import jax.numpy as jnp
from jax import lax
from jax.experimental import pallas as pl
from jax.experimental.pallas import tpu as pltpu
from jax.experimental.pallas import tpu_sc as plsc

N = 10000
E = 160000
D = 256
HD = 128
NSC = 2
NTILE = 16
CHUNK = 128

E_PAD = 163840
IDX_ROWS = E_PAD // CHUNK
ROWS_PER_TILE = IDX_ROWS // NTILE
ROWS_PER_TILE_DEG = IDX_ROWS // (NSC * NTILE)

N_ACC = 10240
SLAB = N_ACC // NTILE

_mesh = plsc.VectorSubcoreMesh(core_axis_name="c", subcore_axis_name="s")


@functools.partial(
    pl.kernel,
    mesh=_mesh,
    out_type=jax.ShapeDtypeStruct((NSC * N_ACC, HD), jnp.float32),
    scratch_types=[
        pltpu.VMEM((ROWS_PER_TILE_DEG, CHUNK), jnp.int32),
        pltpu.VMEM((CHUNK, HD), jnp.float32),
        pltpu.VMEM_SHARED((N_ACC, HD), jnp.float32),
        pltpu.SemaphoreType.DMA,
    ],
)
def _deg_kernel(dst_hbm, ones_hbm, zeros_hbm, deg_hbm, idx_v, ones_v, acc_sh,
                sem):
    c = lax.axis_index("c")
    s = lax.axis_index("s")
    pltpu.sync_copy(zeros_hbm, acc_sh.at[pl.ds(s * SLAB, SLAB)])
    pltpu.sync_copy(ones_hbm, ones_v)
    base = c * (NTILE * ROWS_PER_TILE_DEG) + s * ROWS_PER_TILE_DEG
    pltpu.sync_copy(dst_hbm.at[pl.ds(base, ROWS_PER_TILE_DEG)], idx_v)
    plsc.subcore_barrier()
    handles = [
        pltpu.async_copy(ones_v, acc_sh.at[idx_v.at[j]], sem, add=True)
        for j in range(ROWS_PER_TILE_DEG)
    ]
    for h in handles:
        h.wait()
    plsc.subcore_barrier()
    pltpu.sync_copy(
        acc_sh.at[pl.ds(s * SLAB, SLAB)],
        deg_hbm.at[pl.ds(c * N_ACC + s * SLAB, SLAB)],
    )


@functools.partial(
    pl.kernel,
    mesh=_mesh,
    out_type=jax.ShapeDtypeStruct((NSC * N_ACC, HD), jnp.float32),
    scratch_types=[
        pltpu.VMEM((ROWS_PER_TILE // 2, CHUNK), jnp.int32),
        pltpu.VMEM((ROWS_PER_TILE // 2, CHUNK), jnp.int32),
        pltpu.VMEM((2, CHUNK, HD), jnp.float32),
        pltpu.VMEM_SHARED((N_ACC, HD), jnp.float32),
        pltpu.SemaphoreType.DMA,
        pltpu.SemaphoreType.DMA,
        pltpu.SemaphoreType.DMA,
        pltpu.SemaphoreType.DMA,
    ],
)
def _agg_kernel(hp_hbm, srcs_hbm, dst_hbm, zeros_hbm, out_hbm,
                src_v, dst_v, rows_v, acc_sh, gsem0, gsem1, ssem0, ssem1):
    c = lax.axis_index("c")
    s = lax.axis_index("s")
    gsems = (gsem0, gsem1)
    ssems = (ssem0, ssem1)
    zero_h = pltpu.async_copy(zeros_hbm, acc_sh.at[pl.ds(s * SLAB, SLAB)],
                              gsem1)
    n = ROWS_PER_TILE // 2
    for half in range(2):
        base = s * ROWS_PER_TILE + half * n
        pltpu.sync_copy(srcs_hbm.at[pl.ds(c * IDX_ROWS + base, n)], src_v)
        pltpu.sync_copy(dst_hbm.at[pl.ds(base, n)], dst_v)
        if half == 0:
            zero_h.wait()
            plsc.subcore_barrier()
        gh = [None] * n
        sh = [None] * n
        gh[0] = pltpu.async_copy(hp_hbm.at[src_v.at[0]], rows_v.at[0],
                                 gsems[0])
        for j in range(n):
            b = j % 2
            nb = (j + 1) % 2
            if j + 1 < n:
                if j >= 1:
                    sh[j - 1].wait()
                gh[j + 1] = pltpu.async_copy(
                    hp_hbm.at[src_v.at[j + 1]], rows_v.at[nb], gsems[nb])
            gh[j].wait()
            sh[j] = pltpu.async_copy(
                rows_v.at[b], acc_sh.at[dst_v.at[j]], ssems[b], add=True)
        sh[n - 2].wait()
        sh[n - 1].wait()
    plsc.subcore_barrier()
    pltpu.sync_copy(
        acc_sh.at[pl.ds(s * SLAB, SLAB)],
        out_hbm.at[pl.ds(c * N_ACC + s * SLAB, SLAB)],
    )


_R = 2000
_GRID = N // _R


def _full(shape):
    return pl.BlockSpec(shape, lambda i: tuple(0 for _ in shape))


def _t1_body(deg_ref, x_ref, w_ref, hp_ref, dis_ref):
    d = deg_ref[0] + deg_ref[1]
    dis = lax.rsqrt(1.0 + d)
    h = jnp.dot(x_ref[...], w_ref[...], preferred_element_type=jnp.float32)
    hp_ref[0] = h[:, :HD] * dis
    hp_ref[1] = h[:, HD:] * dis
    dis_ref[...] = dis


def _t1_call(deg3, x, W1):
    return pl.pallas_call(
        _t1_body,
        grid=(_GRID,),
        in_specs=[
            pl.BlockSpec((NSC, _R, HD), lambda i: (0, i, 0)),
            pl.BlockSpec((_R, D), lambda i: (i, 0)),
            _full((D, D)),
        ],
        out_specs=[
            pl.BlockSpec((NSC, _R, HD), lambda i: (0, i, 0)),
            pl.BlockSpec((_R, HD), lambda i: (i, 0)),
        ],
        out_shape=[
            jax.ShapeDtypeStruct((NSC, N, HD), jnp.float32),
            jax.ShapeDtypeStruct((N, HD), jnp.float32),
        ],
    )(deg3, x, W1)


def _ln(t, g, b):
    mu = jnp.mean(t, axis=-1, keepdims=True)
    var = jnp.mean((t - mu) ** 2, axis=-1, keepdims=True)
    return (t - mu) * lax.rsqrt(var + 1e-5) * g + b


def _tmid_body(agg_ref, hp_ref, dis_ref, b_ref, g_ref, be_ref, w_ref, out_ref):
    dis = dis_ref[...]
    b = b_ref[...]
    clo = dis * (agg_ref[0] + hp_ref[0]) + b[:, :HD]
    chi = dis * (agg_ref[1] + hp_ref[1]) + b[:, HD:]
    t = jnp.maximum(jnp.concatenate([clo, chi], axis=-1), 0.0)
    y = _ln(t, g_ref[...], be_ref[...])
    h = jnp.dot(y, w_ref[...], preferred_element_type=jnp.float32)
    out_ref[0] = h[:, :HD] * dis
    out_ref[1] = h[:, HD:] * dis


def _tmid_call(agg3, hp3, dis, b, g, be, W):
    return pl.pallas_call(
        _tmid_body,
        grid=(_GRID,),
        in_specs=[
            pl.BlockSpec((NSC, _R, HD), lambda i: (0, i, 0)),
            pl.BlockSpec((NSC, _R, HD), lambda i: (0, i, 0)),
            pl.BlockSpec((_R, HD), lambda i: (i, 0)),
            _full((1, D)),
            _full((1, D)),
            _full((1, D)),
            _full((D, D)),
        ],
        out_specs=pl.BlockSpec((NSC, _R, HD), lambda i: (0, i, 0)),
        out_shape=jax.ShapeDtypeStruct((NSC, N, HD), jnp.float32),
    )(agg3, hp3, dis, b, g, be, W)


def _t4_body(agg_ref, hp_ref, dis_ref, b_ref, pw1_ref, pb1_ref, pw2_ref,
             pb2_ref, emb_ref, out_ref):
    dis = dis_ref[...]
    b = b_ref[...]
    elo = dis * (agg_ref[0] + hp_ref[0]) + b[:, :HD]
    ehi = dis * (agg_ref[1] + hp_ref[1]) + b[:, HD:]
    emb = jnp.concatenate([elo, ehi], axis=-1)
    emb_ref[...] = emb
    r = jnp.maximum(emb, 0.0)
    p = jnp.dot(r, pw1_ref[...], preferred_element_type=jnp.float32) + pb1_ref[...]
    q = jnp.dot(p, pw2_ref[...], preferred_element_type=jnp.float32) + pb2_ref[...]
    m = jnp.max(q, axis=-1, keepdims=True)
    lse = m + jnp.log(jnp.sum(jnp.exp(q - m), axis=-1, keepdims=True))
    out_ref[...] = q - lse


def _t4_call(agg3, hp3, dis, b3, pW1, pb1, pW2, pb2):
    d_out = pW2.shape[1]
    return pl.pallas_call(
        _t4_body,
        grid=(_GRID,),
        in_specs=[
            pl.BlockSpec((NSC, _R, HD), lambda i: (0, i, 0)),
            pl.BlockSpec((NSC, _R, HD), lambda i: (0, i, 0)),
            pl.BlockSpec((_R, HD), lambda i: (i, 0)),
            _full((1, D)),
            _full((D, D)),
            _full((1, D)),
            _full((D, d_out)),
            _full((1, d_out)),
        ],
        out_specs=[
            pl.BlockSpec((_R, D), lambda i: (i, 0)),
            pl.BlockSpec((_R, d_out), lambda i: (i, 0)),
        ],
        out_shape=[
            jax.ShapeDtypeStruct((N, D), jnp.float32),
            jax.ShapeDtypeStruct((N, d_out), jnp.float32),
        ],
    )(agg3, hp3, dis, b3, pW1, pb1, pW2, pb2)


def kernel(x, edge_index, W1, b1, W2, b2, W3, b3,
           ln1_g, ln1_b, ln2_g, ln2_b, pW1, pb1, pW2, pb2):
    src = edge_index[0].astype(jnp.int32)
    dst = edge_index[1].astype(jnp.int32)
    npad = E_PAD - E
    pad_ids = jnp.arange(npad, dtype=jnp.int32)
    src_p = jnp.concatenate([src, pad_ids % N])
    dst_p = jnp.concatenate([dst, N + (pad_ids % 16)])
    src2d = src_p.reshape(IDX_ROWS, CHUNK)
    dst2d = dst_p.reshape(IDX_ROWS, CHUNK)
    srcs2d = jnp.concatenate([src2d, src2d + N], axis=0)

    ones_rows = jnp.ones((CHUNK, HD), jnp.float32)
    zeros_slab = jnp.zeros((SLAB, HD), jnp.float32)

    b1r = b1.reshape(1, D)
    b2r = b2.reshape(1, D)
    b3r = b3.reshape(1, D)
    g1r = ln1_g.reshape(1, D)
    be1r = ln1_b.reshape(1, D)
    g2r = ln2_g.reshape(1, D)
    be2r = ln2_b.reshape(1, D)
    pb1r = pb1.reshape(1, D)
    pb2r = pb2.reshape(1, pW2.shape[1])

    deg_flat = _deg_kernel(dst2d, ones_rows, zeros_slab)
    deg3 = deg_flat.reshape(NSC, N_ACC, HD)

    hp1, dis = _t1_call(deg3, x, W1)
    agg1 = _agg_kernel(hp1.reshape(NSC * N, HD), srcs2d, dst2d, zeros_slab)
    agg1 = agg1.reshape(NSC, N_ACC, HD)

    hp2 = _tmid_call(agg1, hp1, dis, b1r, g1r, be1r, W2)
    agg2 = _agg_kernel(hp2.reshape(NSC * N, HD), srcs2d, dst2d, zeros_slab)
    agg2 = agg2.reshape(NSC, N_ACC, HD)

    hp3 = _tmid_call(agg2, hp2, dis, b2r, g2r, be2r, W3)
    agg3 = _agg_kernel(hp3.reshape(NSC * N, HD), srcs2d, dst2d, zeros_slab)
    agg3 = agg3.reshape(NSC, N_ACC, HD)

    emb, out = _t4_call(agg3, hp3, dis, b3r, pW1, pb1r, pW2, pb2r)
    return (emb, out)


# speedup vs baseline: 1.0584x; 1.0584x over previous
import functools

import jax
import jax.numpy as jnp
from jax import lax
from jax.experimental import pallas as pl
from jax.experimental.pallas import tpu as pltpu
from jax.experimental.pallas import tpu_sc as plsc

N = 10000
E = 160000
D = 256
HD = 128
NSC = 2
NTILE = 16
CHUNK = 128

E_PAD = 163840
IDX_ROWS = E_PAD // CHUNK
ROWS_PER_TILE = IDX_ROWS // NTILE
ROWS_PER_TILE_DEG = IDX_ROWS // (NSC * NTILE)

N_ACC = 10240
SLAB = N_ACC // NTILE

_mesh = plsc.VectorSubcoreMesh(core_axis_name="c", subcore_axis_name="s")


@functools.partial(
    pl.kernel,
    mesh=_mesh,
    out_type=jax.ShapeDtypeStruct((NSC * N_ACC, HD), jnp.float32),
    scratch_types=[
        pltpu.VMEM((ROWS_PER_TILE_DEG, CHUNK), jnp.int32),
        pltpu.VMEM((CHUNK,), jnp.float32),
        pltpu.VMEM((SLAB,), jnp.float32),
        pltpu.VMEM((SLAB, HD), jnp.float32),
        pltpu.VMEM_SHARED((N_ACC,), jnp.float32),
        pltpu.SemaphoreType.DMA,
    ],
)
def _deg_kernel(dst_hbm, ones_hbm, zeros_hbm, deg_hbm, idx_v, ones_v, cnt_v,
                exp_v, acc_sh, sem):
    c = lax.axis_index("c")
    s = lax.axis_index("s")
    pltpu.sync_copy(zeros_hbm, acc_sh.at[pl.ds(s * SLAB, SLAB)])
    pltpu.sync_copy(ones_hbm, ones_v)
    base = c * (NTILE * ROWS_PER_TILE_DEG) + s * ROWS_PER_TILE_DEG
    pltpu.sync_copy(dst_hbm.at[pl.ds(base, ROWS_PER_TILE_DEG)], idx_v)
    plsc.subcore_barrier()
    handles = [
        pltpu.async_copy(ones_v, acc_sh.at[idx_v.at[j]], sem, add=True)
        for j in range(ROWS_PER_TILE_DEG)
    ]
    for h in handles:
        h.wait()
    plsc.subcore_barrier()
    pltpu.sync_copy(acc_sh.at[pl.ds(s * SLAB, SLAB)], cnt_v)

    ones16 = jnp.ones((16,), jnp.float32)
    for t in range(SLAB // 16):
        v = cnt_v[pl.ds(16 * t, 16)]
        for m in range(16):
            row = v[m] * ones16
            for k in range(HD // 16):
                exp_v[16 * t + m, pl.ds(16 * k, 16)] = row
    pltpu.sync_copy(exp_v, deg_hbm.at[pl.ds(c * N_ACC + s * SLAB, SLAB)])


@functools.partial(
    pl.kernel,
    mesh=_mesh,
    out_type=jax.ShapeDtypeStruct((NSC * N_ACC, HD), jnp.float32),
    scratch_types=[
        pltpu.VMEM((ROWS_PER_TILE // 2, CHUNK), jnp.int32),
        pltpu.VMEM((ROWS_PER_TILE // 2, CHUNK), jnp.int32),
        pltpu.VMEM((2, CHUNK, HD), jnp.float32),
        pltpu.VMEM_SHARED((N_ACC, HD), jnp.float32),
        pltpu.SemaphoreType.DMA,
        pltpu.SemaphoreType.DMA,
        pltpu.SemaphoreType.DMA,
        pltpu.SemaphoreType.DMA,
    ],
)
def _agg_kernel(hp_hbm, srcs_hbm, dst_hbm, zeros_hbm, out_hbm,
                src_v, dst_v, rows_v, acc_sh, gsem0, gsem1, ssem0, ssem1):
    c = lax.axis_index("c")
    s = lax.axis_index("s")
    gsems = (gsem0, gsem1)
    ssems = (ssem0, ssem1)
    zero_h = pltpu.async_copy(zeros_hbm, acc_sh.at[pl.ds(s * SLAB, SLAB)],
                              gsem1)
    n = ROWS_PER_TILE // 2
    for half in range(2):
        base = s * ROWS_PER_TILE + half * n
        pltpu.sync_copy(srcs_hbm.at[pl.ds(c * IDX_ROWS + base, n)], src_v)
        pltpu.sync_copy(dst_hbm.at[pl.ds(base, n)], dst_v)
        if half == 0:
            zero_h.wait()
            plsc.subcore_barrier()
        gh = [None] * n
        sh = [None] * n
        gh[0] = pltpu.async_copy(hp_hbm.at[src_v.at[0]], rows_v.at[0],
                                 gsems[0])
        for j in range(n):
            b = j % 2
            nb = (j + 1) % 2
            if j + 1 < n:
                if j >= 1:
                    sh[j - 1].wait()
                gh[j + 1] = pltpu.async_copy(
                    hp_hbm.at[src_v.at[j + 1]], rows_v.at[nb], gsems[nb])
            gh[j].wait()
            sh[j] = pltpu.async_copy(
                rows_v.at[b], acc_sh.at[dst_v.at[j]], ssems[b], add=True)
        sh[n - 2].wait()
        sh[n - 1].wait()
    plsc.subcore_barrier()
    pltpu.sync_copy(
        acc_sh.at[pl.ds(s * SLAB, SLAB)],
        out_hbm.at[pl.ds(c * N_ACC + s * SLAB, SLAB)],
    )


_R = 2000
_GRID = N // _R


def _full(shape):
    return pl.BlockSpec(shape, lambda i: tuple(0 for _ in shape))


def _t1_body(deg_ref, x_ref, w_ref, hp_ref, dis_ref):
    d = deg_ref[0] + deg_ref[1]
    dis = lax.rsqrt(1.0 + d)
    h = jnp.dot(x_ref[...], w_ref[...], preferred_element_type=jnp.float32)
    hp_ref[0] = h[:, :HD] * dis
    hp_ref[1] = h[:, HD:] * dis
    dis_ref[...] = dis


def _t1_call(deg3, x, W1):
    return pl.pallas_call(
        _t1_body,
        grid=(_GRID,),
        in_specs=[
            pl.BlockSpec((NSC, _R, HD), lambda i: (0, i, 0)),
            pl.BlockSpec((_R, D), lambda i: (i, 0)),
            _full((D, D)),
        ],
        out_specs=[
            pl.BlockSpec((NSC, _R, HD), lambda i: (0, i, 0)),
            pl.BlockSpec((_R, HD), lambda i: (i, 0)),
        ],
        out_shape=[
            jax.ShapeDtypeStruct((NSC, N, HD), jnp.float32),
            jax.ShapeDtypeStruct((N, HD), jnp.float32),
        ],
    )(deg3, x, W1)


def _ln(t, g, b):
    mu = jnp.mean(t, axis=-1, keepdims=True)
    var = jnp.mean((t - mu) ** 2, axis=-1, keepdims=True)
    return (t - mu) * lax.rsqrt(var + 1e-5) * g + b


def _tmid_body(agg_ref, hp_ref, dis_ref, b_ref, g_ref, be_ref, w_ref, out_ref):
    dis = dis_ref[...]
    b = b_ref[...]
    clo = dis * (agg_ref[0] + hp_ref[0]) + b[:, :HD]
    chi = dis * (agg_ref[1] + hp_ref[1]) + b[:, HD:]
    t = jnp.maximum(jnp.concatenate([clo, chi], axis=-1), 0.0)
    y = _ln(t, g_ref[...], be_ref[...])
    h = jnp.dot(y, w_ref[...], preferred_element_type=jnp.float32)
    out_ref[0] = h[:, :HD] * dis
    out_ref[1] = h[:, HD:] * dis


def _tmid_call(agg3, hp3, dis, b, g, be, W):
    return pl.pallas_call(
        _tmid_body,
        grid=(_GRID,),
        in_specs=[
            pl.BlockSpec((NSC, _R, HD), lambda i: (0, i, 0)),
            pl.BlockSpec((NSC, _R, HD), lambda i: (0, i, 0)),
            pl.BlockSpec((_R, HD), lambda i: (i, 0)),
            _full((1, D)),
            _full((1, D)),
            _full((1, D)),
            _full((D, D)),
        ],
        out_specs=pl.BlockSpec((NSC, _R, HD), lambda i: (0, i, 0)),
        out_shape=jax.ShapeDtypeStruct((NSC, N, HD), jnp.float32),
    )(agg3, hp3, dis, b, g, be, W)


def _t4_body(agg_ref, hp_ref, dis_ref, b_ref, pw1_ref, pb1_ref, pw2_ref,
             pb2_ref, emb_ref, out_ref):
    dis = dis_ref[...]
    b = b_ref[...]
    elo = dis * (agg_ref[0] + hp_ref[0]) + b[:, :HD]
    ehi = dis * (agg_ref[1] + hp_ref[1]) + b[:, HD:]
    emb = jnp.concatenate([elo, ehi], axis=-1)
    emb_ref[...] = emb
    r = jnp.maximum(emb, 0.0)
    p = jnp.dot(r, pw1_ref[...], preferred_element_type=jnp.float32) + pb1_ref[...]
    q = jnp.dot(p, pw2_ref[...], preferred_element_type=jnp.float32) + pb2_ref[...]
    m = jnp.max(q, axis=-1, keepdims=True)
    lse = m + jnp.log(jnp.sum(jnp.exp(q - m), axis=-1, keepdims=True))
    out_ref[...] = q - lse


def _t4_call(agg3, hp3, dis, b3, pW1, pb1, pW2, pb2):
    d_out = pW2.shape[1]
    return pl.pallas_call(
        _t4_body,
        grid=(_GRID,),
        in_specs=[
            pl.BlockSpec((NSC, _R, HD), lambda i: (0, i, 0)),
            pl.BlockSpec((NSC, _R, HD), lambda i: (0, i, 0)),
            pl.BlockSpec((_R, HD), lambda i: (i, 0)),
            _full((1, D)),
            _full((D, D)),
            _full((1, D)),
            _full((D, d_out)),
            _full((1, d_out)),
        ],
        out_specs=[
            pl.BlockSpec((_R, D), lambda i: (i, 0)),
            pl.BlockSpec((_R, d_out), lambda i: (i, 0)),
        ],
        out_shape=[
            jax.ShapeDtypeStruct((N, D), jnp.float32),
            jax.ShapeDtypeStruct((N, d_out), jnp.float32),
        ],
    )(agg3, hp3, dis, b3, pW1, pb1, pW2, pb2)


def kernel(x, edge_index, W1, b1, W2, b2, W3, b3,
           ln1_g, ln1_b, ln2_g, ln2_b, pW1, pb1, pW2, pb2):
    src = edge_index[0].astype(jnp.int32)
    dst = edge_index[1].astype(jnp.int32)
    npad = E_PAD - E
    pad_ids = jnp.arange(npad, dtype=jnp.int32)
    src_p = jnp.concatenate([src, pad_ids % N])
    dst_p = jnp.concatenate([dst, N + (pad_ids % 16)])
    src2d = src_p.reshape(IDX_ROWS, CHUNK)
    dst2d = dst_p.reshape(IDX_ROWS, CHUNK)
    srcs2d = jnp.concatenate([src2d, src2d + N], axis=0)

    ones_1d = jnp.ones((CHUNK,), jnp.float32)
    zeros_1d = jnp.zeros((SLAB,), jnp.float32)
    zeros_slab = jnp.zeros((SLAB, HD), jnp.float32)

    b1r = b1.reshape(1, D)
    b2r = b2.reshape(1, D)
    b3r = b3.reshape(1, D)
    g1r = ln1_g.reshape(1, D)
    be1r = ln1_b.reshape(1, D)
    g2r = ln2_g.reshape(1, D)
    be2r = ln2_b.reshape(1, D)
    pb1r = pb1.reshape(1, D)
    pb2r = pb2.reshape(1, pW2.shape[1])

    deg_flat = _deg_kernel(dst2d, ones_1d, zeros_1d)
    deg3 = deg_flat.reshape(NSC, N_ACC, HD)

    hp1, dis = _t1_call(deg3, x, W1)
    agg1 = _agg_kernel(hp1.reshape(NSC * N, HD), srcs2d, dst2d, zeros_slab)
    agg1 = agg1.reshape(NSC, N_ACC, HD)

    hp2 = _tmid_call(agg1, hp1, dis, b1r, g1r, be1r, W2)
    agg2 = _agg_kernel(hp2.reshape(NSC * N, HD), srcs2d, dst2d, zeros_slab)
    agg2 = agg2.reshape(NSC, N_ACC, HD)

    hp3 = _tmid_call(agg2, hp2, dis, b2r, g2r, be2r, W3)
    agg3 = _agg_kernel(hp3.reshape(NSC * N, HD), srcs2d, dst2d, zeros_slab)
    agg3 = agg3.reshape(NSC, N_ACC, HD)

    emb, out = _t4_call(agg3, hp3, dis, b3r, pW1, pb1r, pW2, pb2r)
    return (emb, out)
